# baseline (speedup 1.0000x reference)
import functools

import jax
import jax.numpy as jnp
from jax import lax
from jax.experimental import pallas as pl
from jax.experimental.pallas import tpu as pltpu
from jax.experimental.pallas import tpu_sc as plsc

DEPTH = 64
NC, NS = 2, 16
NW = NC * NS
IDX_LANE = 128
K = 8
CHUNK = K * IDX_LANE


@functools.lru_cache(maxsize=None)
def _build(n_rows):
    assert n_rows % (NW * CHUNK) == 0
    n_chunks = n_rows // (NW * CHUNK)
    mesh = plsc.VectorSubcoreMesh(core_axis_name="c", subcore_axis_name="s")

    @functools.partial(
        pl.kernel,
        mesh=mesh,
        out_type=jax.ShapeDtypeStruct((n_rows, DEPTH), jnp.float32),
        scratch_types=[
            pltpu.VMEM((K, IDX_LANE), jnp.int32),
            pltpu.VMEM((CHUNK, DEPTH), jnp.float32),
            pltpu.SemaphoreType.DMA,
        ],
        compiler_params=pltpu.CompilerParams(use_tc_tiling_on_sc=False),
    )
    def body(table_hbm, idx_hbm, out_hbm, idx_v, rows_v, sem):
        wid = lax.axis_index("s") * NC + lax.axis_index("c")

        def chunk_body(i, carry):
            row_base = (wid * n_chunks + i) * K
            pltpu.sync_copy(idx_hbm.at[pl.ds(row_base, K)], idx_v)
            cps = []
            for j in range(K):
                cps.append(
                    pltpu.async_copy(
                        table_hbm.at[idx_v.at[j]],
                        rows_v.at[pl.ds(j * IDX_LANE, IDX_LANE)],
                        sem,
                    )
                )
            for cp in cps:
                cp.wait()
            pltpu.sync_copy(
                rows_v, out_hbm.at[pl.ds(row_base * IDX_LANE, CHUNK)]
            )
            return carry

        lax.fori_loop(0, n_chunks, chunk_body, 0)

    return body


def kernel(texts_tokenized, embedding_table):
    b, s = texts_tokenized.shape
    n_rows = b * s
    idx = texts_tokenized.reshape(n_rows // IDX_LANE, IDX_LANE).astype(jnp.int32)
    out = _build(n_rows)(embedding_table, idx)
    return out.reshape(b, s, DEPTH)


# speedup vs baseline: 1.0021x; 1.0021x over previous
import functools

import jax
import jax.numpy as jnp
from jax import lax
from jax.experimental import pallas as pl
from jax.experimental.pallas import tpu as pltpu
from jax.experimental.pallas import tpu_sc as plsc

DEPTH = 64
NC, NS = 2, 16
NW = NC * NS
IDX_LANE = 128
K = 4
CHUNK = K * IDX_LANE
NBUF = 2


@functools.lru_cache(maxsize=None)
def _build(n_rows):
    assert n_rows % (NW * CHUNK) == 0
    n_chunks = n_rows // (NW * CHUNK)
    assert n_chunks % 2 == 0 and n_chunks >= 4
    mesh = plsc.VectorSubcoreMesh(core_axis_name="c", subcore_axis_name="s")

    @functools.partial(
        pl.kernel,
        mesh=mesh,
        out_type=jax.ShapeDtypeStruct((n_rows, DEPTH), jnp.float32),
        scratch_types=[
            pltpu.VMEM((NBUF, K, IDX_LANE), jnp.int32),
            pltpu.VMEM((NBUF, CHUNK, DEPTH), jnp.float32),
            [pltpu.SemaphoreType.DMA] * NBUF,
            [pltpu.SemaphoreType.DMA] * NBUF,
        ],
        compiler_params=pltpu.CompilerParams(use_tc_tiling_on_sc=False),
    )
    def body(table_hbm, idx_hbm, out_hbm, idx_v, rows_v, sem_g, sem_o):
        wid = lax.axis_index("s") * NC + lax.axis_index("c")
        chunk0 = wid * n_chunks

        def load_idx(i, b):
            pltpu.sync_copy(
                idx_hbm.at[pl.ds((chunk0 + i) * K, K)], idx_v.at[b]
            )

        def gather(b):
            return [
                pltpu.make_async_copy(
                    table_hbm.at[idx_v.at[b, j]],
                    rows_v.at[b, pl.ds(j * IDX_LANE, IDX_LANE)],
                    sem_g[b],
                )
                for j in range(K)
            ]

        def fire_g(b):
            for cp in gather(b):
                cp.start()

        def wait_g(b):
            for cp in gather(b):
                cp.wait()

        def store(i, b):
            return pltpu.make_async_copy(
                rows_v.at[b],
                out_hbm.at[pl.ds((chunk0 + i) * CHUNK, CHUNK)],
                sem_o[b],
            )

        load_idx(0, 0)
        fire_g(0)
        wait_g(0)
        store(0, 0).start()
        load_idx(1, 1)
        fire_g(1)

        def outer(g, carry):
            for b in (1, 0):
                i = 1 + g * 2 + (1 - b)
                ob = 1 - b
                wait_g(b)
                store(i, b).start()
                load_idx(i + 1, ob)
                store(i - 1, ob).wait()
                fire_g(ob)
            return carry

        lax.fori_loop(0, (n_chunks - 2) // 2, outer, 0)

        i_last = n_chunks - 1
        wait_g(1)
        store(i_last, 1).start()
        store(i_last - 1, 0).wait()
        store(i_last, 1).wait()

    return body


def kernel(texts_tokenized, embedding_table):
    b, s = texts_tokenized.shape
    n_rows = b * s
    idx = texts_tokenized.reshape(n_rows // IDX_LANE, IDX_LANE).astype(jnp.int32)
    out = _build(n_rows)(embedding_table, idx)
    return out.reshape(b, s, DEPTH)


# speedup vs baseline: 1.0082x; 1.0062x over previous
import functools

import jax
import jax.numpy as jnp
from jax import lax
from jax.experimental import pallas as pl
from jax.experimental.pallas import tpu as pltpu
from jax.experimental.pallas import tpu_sc as plsc

DEPTH = 64
NC, NS = 2, 16
NW = NC * NS
GMAX = 128
T = 4
NBUF = 2


@functools.lru_cache(maxsize=None)
def _build(n_texts, seq_len):
    assert n_texts % (NW * T) == 0
    n_chunks = n_texts // (NW * T)
    assert n_chunks % 2 == 0 and n_chunks >= 4
    pieces = []
    off = 0
    while off < seq_len:
        n = min(GMAX, seq_len - off)
        pieces.append((off, n))
        off += n
    mesh = plsc.VectorSubcoreMesh(core_axis_name="c", subcore_axis_name="s")

    @functools.partial(
        pl.kernel,
        mesh=mesh,
        out_type=jax.ShapeDtypeStruct((n_texts, seq_len, DEPTH), jnp.float32),
        scratch_types=[
            pltpu.VMEM((NBUF, T, seq_len), jnp.int32),
            pltpu.VMEM((NBUF, T, seq_len, DEPTH), jnp.float32),
            [pltpu.SemaphoreType.DMA] * NBUF,
            [pltpu.SemaphoreType.DMA] * NBUF,
        ],
        compiler_params=pltpu.CompilerParams(use_tc_tiling_on_sc=False),
    )
    def body(table_hbm, idx_hbm, out_hbm, idx_v, rows_v, sem_g, sem_o):
        wid = lax.axis_index("s") * NC + lax.axis_index("c")
        text0 = wid * n_chunks * T

        def load_idx(i, b):
            pltpu.sync_copy(
                idx_hbm.at[pl.ds(text0 + i * T, T)], idx_v.at[b]
            )

        def gather(b):
            cps = []
            for t in range(T):
                for off, n in pieces:
                    cps.append(
                        pltpu.make_async_copy(
                            table_hbm.at[idx_v.at[b, t, pl.ds(off, n)]],
                            rows_v.at[b, t, pl.ds(off, n)],
                            sem_g[b],
                        )
                    )
            return cps

        def fire_g(b):
            for cp in gather(b):
                cp.start()

        def wait_g(b):
            for cp in gather(b):
                cp.wait()

        def store(i, b):
            return pltpu.make_async_copy(
                rows_v.at[b],
                out_hbm.at[pl.ds(text0 + i * T, T)],
                sem_o[b],
            )

        load_idx(0, 0)
        fire_g(0)
        wait_g(0)
        store(0, 0).start()
        load_idx(1, 1)
        fire_g(1)

        def outer(g, carry):
            for b in (1, 0):
                i = 1 + g * 2 + (1 - b)
                ob = 1 - b
                wait_g(b)
                store(i, b).start()
                load_idx(i + 1, ob)
                store(i - 1, ob).wait()
                fire_g(ob)
            return carry

        lax.fori_loop(0, (n_chunks - 2) // 2, outer, 0)

        i_last = n_chunks - 1
        wait_g(1)
        store(i_last, 1).start()
        store(i_last - 1, 0).wait()
        store(i_last, 1).wait()

    return body


def kernel(texts_tokenized, embedding_table):
    n_texts, seq_len = texts_tokenized.shape
    return _build(n_texts, seq_len)(
        embedding_table, texts_tokenized.astype(jnp.int32)
    )


# speedup vs baseline: 1.1997x; 1.1899x over previous
import functools

import jax
import jax.numpy as jnp
from jax import lax
from jax.experimental import pallas as pl
from jax.experimental.pallas import tpu as pltpu
from jax.experimental.pallas import tpu_sc as plsc

DEPTH = 64
WIDE = 2 * DEPTH
NC, NS = 2, 16
NW = NC * NS
IDX_LANE = 128
K = 2
CHUNK = K * IDX_LANE
NBUF = 2


@functools.lru_cache(maxsize=None)
def _build(n_rows):
    assert n_rows % (NW * CHUNK) == 0
    n_chunks = n_rows // (NW * CHUNK)
    assert n_chunks % 2 == 0 and n_chunks >= 4
    mesh = plsc.VectorSubcoreMesh(core_axis_name="c", subcore_axis_name="s")

    @functools.partial(
        pl.kernel,
        mesh=mesh,
        out_type=jax.ShapeDtypeStruct((n_rows, WIDE), jnp.float32),
        scratch_types=[
            pltpu.VMEM((NBUF, K, IDX_LANE), jnp.int32),
            pltpu.VMEM((NBUF, CHUNK, WIDE), jnp.float32),
            [pltpu.SemaphoreType.DMA] * NBUF,
            [pltpu.SemaphoreType.DMA] * NBUF,
        ],
        compiler_params=pltpu.CompilerParams(use_tc_tiling_on_sc=True),
    )
    def body(table_hbm, idx_hbm, out_hbm, idx_v, rows_v, sem_g, sem_o):
        wid = lax.axis_index("s") * NC + lax.axis_index("c")
        chunk0 = wid * n_chunks

        def load_idx(i, b):
            pltpu.sync_copy(
                idx_hbm.at[pl.ds((chunk0 + i) * K, K)], idx_v.at[b]
            )

        def gather(b):
            return [
                pltpu.make_async_copy(
                    table_hbm.at[idx_v.at[b, j]],
                    rows_v.at[b, pl.ds(j * IDX_LANE, IDX_LANE)],
                    sem_g[b],
                )
                for j in range(K)
            ]

        def fire_g(b):
            for cp in gather(b):
                cp.start()

        def wait_g(b):
            for cp in gather(b):
                cp.wait()

        def store(i, b):
            return pltpu.make_async_copy(
                rows_v.at[b],
                out_hbm.at[pl.ds((chunk0 + i) * CHUNK, CHUNK)],
                sem_o[b],
            )

        load_idx(0, 0)
        fire_g(0)
        wait_g(0)
        store(0, 0).start()
        load_idx(1, 1)
        fire_g(1)

        def outer(g, carry):
            for b in (1, 0):
                i = 1 + g * 2 + (1 - b)
                ob = 1 - b
                wait_g(b)
                store(i, b).start()
                load_idx(i + 1, ob)
                store(i - 1, ob).wait()
                fire_g(ob)
            return carry

        lax.fori_loop(0, (n_chunks - 2) // 2, outer, 0)

        i_last = n_chunks - 1
        wait_g(1)
        store(i_last, 1).start()
        store(i_last - 1, 0).wait()
        store(i_last, 1).wait()

    return body


def kernel(texts_tokenized, embedding_table):
    b, s = texts_tokenized.shape
    n_rows = b * s
    idx = texts_tokenized.reshape(n_rows // IDX_LANE, IDX_LANE).astype(jnp.int32)
    table_p = jnp.pad(embedding_table, ((0, 0), (0, WIDE - DEPTH)))
    out = _build(n_rows)(table_p, idx)
    return out[:, :DEPTH].reshape(b, s, DEPTH)


# speedup vs baseline: 1.2396x; 1.0333x over previous
import functools

import jax
import jax.numpy as jnp
from jax import lax
from jax.experimental import pallas as pl
from jax.experimental.pallas import tpu as pltpu
from jax.experimental.pallas import tpu_sc as plsc

DEPTH = 64
WIDE = 2 * DEPTH
NC, NS = 2, 16
NW = NC * NS
IDX_LANE = 128
K = 2
CHUNK = K * IDX_LANE
NBUF = 3


@functools.lru_cache(maxsize=None)
def _build(n_rows):
    assert n_rows % (NW * CHUNK) == 0
    n_chunks = n_rows // (NW * CHUNK)
    assert (n_chunks - 4) % NBUF == 0 and n_chunks >= 8
    idx_rows = n_chunks * K
    mesh = plsc.VectorSubcoreMesh(core_axis_name="c", subcore_axis_name="s")

    @functools.partial(
        pl.kernel,
        mesh=mesh,
        out_type=jax.ShapeDtypeStruct((n_rows, WIDE), jnp.float32),
        scratch_types=[
            pltpu.VMEM((idx_rows, IDX_LANE), jnp.int32),
            pltpu.VMEM((NBUF, CHUNK, WIDE), jnp.float32),
            [pltpu.SemaphoreType.DMA] * NBUF,
            [pltpu.SemaphoreType.DMA] * NBUF,
        ],
        compiler_params=pltpu.CompilerParams(use_tc_tiling_on_sc=True),
    )
    def body(table_hbm, idx_hbm, out_hbm, idx_v, rows_v, sem_g, sem_o):
        wid = lax.axis_index("s") * NC + lax.axis_index("c")
        chunk0 = wid * n_chunks

        pltpu.sync_copy(idx_hbm.at[pl.ds(chunk0 * K, idx_rows)], idx_v)

        def gather(i, b):
            return [
                pltpu.make_async_copy(
                    table_hbm.at[idx_v.at[i * K + j]],
                    rows_v.at[b, pl.ds(j * IDX_LANE, IDX_LANE)],
                    sem_g[b],
                )
                for j in range(K)
            ]

        def fire_g(i, b):
            for cp in gather(i, b):
                cp.start()

        def wait_g(i, b):
            for cp in gather(i, b):
                cp.wait()

        def store(i, b):
            return pltpu.make_async_copy(
                rows_v.at[b],
                out_hbm.at[pl.ds((chunk0 + i) * CHUNK, CHUNK)],
                sem_o[b],
            )

        fire_g(0, 0)
        fire_g(1, 1)
        wait_g(0, 0)
        store(0, 0).start()
        fire_g(2, 2)
        wait_g(1, 1)
        store(1, 1).start()
        store(0, 0).wait()
        fire_g(3, 0)

        def outer(g, carry):
            for k in range(NBUF):
                i = 2 + g * NBUF + k
                b = (2 + k) % NBUF
                wait_g(i, b)
                store(i, b).start()
                store(i - 1, (b + NBUF - 1) % NBUF).wait()
                fire_g(i + 2, (b + 2) % NBUF)
            return carry

        lax.fori_loop(0, (n_chunks - 4) // NBUF, outer, 0)

        i1, i2 = n_chunks - 2, n_chunks - 1
        b1, b2 = i1 % NBUF, i2 % NBUF
        wait_g(i1, b1)
        store(i1, b1).start()
        store(i1 - 1, (b1 + NBUF - 1) % NBUF).wait()
        wait_g(i2, b2)
        store(i2, b2).start()
        store(i1, b1).wait()
        store(i2, b2).wait()

    return body


def kernel(texts_tokenized, embedding_table):
    b, s = texts_tokenized.shape
    n_rows = b * s
    idx = texts_tokenized.reshape(n_rows // IDX_LANE, IDX_LANE).astype(jnp.int32)
    table_p = jnp.pad(embedding_table, ((0, 0), (0, WIDE - DEPTH)))
    out = _build(n_rows)(table_p, idx)
    return out[:, :DEPTH].reshape(b, s, DEPTH)
